# baseline (speedup 1.0000x reference)
import functools

import jax
import jax.numpy as jnp
from jax import lax
from jax.experimental import pallas as pl
from jax.experimental.pallas import tpu as pltpu
from jax.experimental.pallas import tpu_sc as plsc

B = 4096
S = 16
D = 512
E = 1024
C = 40
NNODES = 50000

NC = 2
NS = 16
NW = NC * NS
CH = D // 128
L = 16

SEG = 128
ZB = 32


def _sc_body(bpw, nk, acc, zps,
             nseg_hbm, ndst_hbm, sseg_hbm, feat_hbm, self_out, agg_out,
             nidx_v, ndst_v, sidx_v, rows_v, zeros_v, acc_sh, sem0, sem1):
    cid = lax.axis_index("c")
    sid = lax.axis_index("s")
    wid = cid * NS + sid
    base4 = pl.multiple_of(wid * (bpw * CH), bpw * CH)
    nb = pl.multiple_of(wid * (bpw * S * CH), bpw * S * CH)

    pltpu.sync_copy(nseg_hbm.at[pl.ds(nb, bpw * S * CH)], nidx_v)
    pltpu.sync_copy(ndst_hbm.at[pl.ds(nb, bpw * S * CH)], ndst_v)
    pltpu.sync_copy(sseg_hbm.at[pl.ds(base4, bpw * CH)], sidx_v)

    for t in range(ZB):
        for u in range(8):
            zeros_v[t, pl.ds(u * L, L)] = jnp.zeros((L,), jnp.float32)
    zbase = pl.multiple_of(sid * zps, zps)
    for m in range(zps // ZB):
        pltpu.sync_copy(zeros_v, acc_sh.at[pl.ds(zbase + m * ZB, ZB)])
    plsc.subcore_barrier()

    sems = (sem0, sem1)

    def nstart(k, slot):
        pltpu.async_copy(feat_hbm.at[nidx_v.at[pl.ds(k * SEG, SEG)]],
                         rows_v.at[pl.ds(slot * SEG, SEG)], sems[slot])

    def nwait(k, slot):
        pltpu.make_async_copy(feat_hbm.at[nidx_v.at[pl.ds(k * SEG, SEG)]],
                              rows_v.at[pl.ds(slot * SEG, SEG)], sems[slot]).wait()

    nstart(0, 0)
    nstart(1, 1)

    def body(i, carry):
        for half in range(2):
            k = 2 * i + half
            nwait(k, half)
            pltpu.sync_copy(rows_v.at[pl.ds(half * SEG, SEG)],
                            acc_sh.at[ndst_v.at[pl.ds(k * SEG, SEG)]],
                            add=True)

            @pl.when(k + 2 < nk)
            def _refill():
                nstart(k + 2, half)
        return carry

    lax.fori_loop(0, nk // 2, body, 0)

    nsp = bpw * CH // SEG

    def sstart(p, slot):
        pltpu.async_copy(feat_hbm.at[sidx_v.at[pl.ds(p * SEG, SEG)]],
                         rows_v.at[pl.ds(slot * SEG, SEG)], sems[slot])

    def swait(p, slot):
        pltpu.make_async_copy(feat_hbm.at[sidx_v.at[pl.ds(p * SEG, SEG)]],
                              rows_v.at[pl.ds(slot * SEG, SEG)], sems[slot]).wait()

    sstart(0, 0)
    if nsp > 1:
        sstart(1, 1)
    for p in range(nsp):
        slot = p % 2
        swait(p, slot)
        pltpu.sync_copy(rows_v.at[pl.ds(slot * SEG, SEG)],
                        self_out.at[pl.ds(base4 + p * SEG, SEG)])
        if p + 2 < nsp:
            sstart(p + 2, slot)

    plsc.subcore_barrier()
    for m in range(zps // SEG):
        off = zbase + m * SEG
        pltpu.sync_copy(acc_sh.at[pl.ds(off, SEG)],
                        agg_out.at[pl.ds(cid * acc + off, SEG)])


@functools.cache
def _make_sc_gather(b):
    bpw = b // NW
    nk = bpw * S * CH // SEG
    acc = b * CH // NC
    zps = acc // NS
    body = functools.partial(_sc_body, bpw, nk, acc, zps)
    return pl.kernel(
        body,
        out_type=[
            jax.ShapeDtypeStruct((b * CH, 128), jnp.float32),
            jax.ShapeDtypeStruct((b * CH, 128), jnp.float32),
        ],
        mesh=plsc.VectorSubcoreMesh(core_axis_name="c", subcore_axis_name="s",
                                    num_cores=NC, num_subcores=NS),
        scratch_types=[
            pltpu.VMEM((bpw * S * CH,), jnp.int32),
            pltpu.VMEM((bpw * S * CH,), jnp.int32),
            pltpu.VMEM((bpw * CH,), jnp.int32),
            pltpu.VMEM((2 * SEG, 128), jnp.float32),
            pltpu.VMEM((ZB, 128), jnp.float32),
            pltpu.VMEM_SHARED((acc, 128), jnp.float32),
            pltpu.SemaphoreType.DMA,
            pltpu.SemaphoreType.DMA,
        ],
        compiler_params=pltpu.CompilerParams(use_tc_tiling_on_sc=False),
    )


def _tc_body(self_ref, agg_ref, w1_ref, w2_ref, wcls_ref, out_ref):
    bb = self_ref.shape[0] // CH

    def logical(ref):
        x = ref[...].reshape(bb // 8, CH, 8, 128)
        return x.transpose(0, 2, 1, 3).reshape(bb, D)

    h = lax.dot_general(logical(self_ref), w1_ref[...],
                        (((1,), (1,)), ((), ())),
                        preferred_element_type=jnp.float32)
    h = h + lax.dot_general(logical(agg_ref), w2_ref[...],
                            (((1,), (1,)), ((), ())),
                            preferred_element_type=jnp.float32)
    h = jnp.maximum(h, 0.0)
    out_ref[...] = lax.dot_general(h, wcls_ref[...],
                                   (((1,), (1,)), ((), ())),
                                   preferred_element_type=jnp.float32)


def _tc_head(self2, agg2, w1, w2, wcls, b, block_b=512):
    grid = (b // block_b,)
    return pl.pallas_call(
        _tc_body,
        grid=grid,
        in_specs=[
            pl.BlockSpec((block_b * CH, 128), lambda i: (i, 0)),
            pl.BlockSpec((block_b * CH, 128), lambda i: (i, 0)),
            pl.BlockSpec((E, D), lambda i: (0, 0)),
            pl.BlockSpec((E, D), lambda i: (0, 0)),
            pl.BlockSpec((C, E), lambda i: (0, 0)),
        ],
        out_specs=pl.BlockSpec((block_b, C), lambda i: (i, 0)),
        out_shape=jax.ShapeDtypeStruct((b, C), jnp.float32),
    )(self2, agg2, w1, w2, wcls)


def _gather_indices(nodes, neigh_idx, b):
    j8 = jnp.arange(CH, dtype=jnp.int32) * 8
    cn = (neigh_idx >> 3) * 32 + (neigh_idx & 7)
    nseg = (cn[:, None, :] + j8[None, :, None]).reshape(-1)
    cs = (nodes >> 3) * 32 + (nodes & 7)
    sseg = (cs.reshape(b // 8, 8)[:, None, :] + j8[None, :, None]).reshape(-1)

    acc = b * CH // NC
    g = jnp.arange(b, dtype=jnp.int32)
    gcr = (g >> 3) * 32 + (g & 7) - (2 * g // b) * acc
    ndst = jnp.broadcast_to((gcr[:, None] + j8[None, :])[:, :, None],
                            (b, CH, S)).reshape(-1)
    return nseg, ndst, sseg


def kernel(nodes, neigh_idx, features, W_enc, weight):
    nodes = nodes.astype(jnp.int32)
    neigh_idx = neigh_idx.astype(jnp.int32)

    feat2 = (features.reshape(NNODES // 8, 8, CH, 128)
             .transpose(0, 2, 1, 3).reshape(NNODES * CH, 128))

    w1 = W_enc[:, :D]
    w2 = W_enc[:, D:] * (1.0 / S)

    hb = B // 4
    sc = _make_sc_gather(hb)
    outs = []
    for h in range(4):
        nh = lax.slice_in_dim(nodes, h * hb, (h + 1) * hb)
        eh = lax.slice_in_dim(neigh_idx, h * hb, (h + 1) * hb)
        nseg, ndst, sseg = _gather_indices(nh, eh, hb)
        self2, agg2 = sc(nseg, ndst, sseg, feat2)
        outs.append(_tc_head(self2, agg2, w1, w2, weight, hb))
    return jnp.concatenate(outs, axis=0)


# speedup vs baseline: 1.0875x; 1.0875x over previous
import functools

import jax
import jax.numpy as jnp
from jax import lax
from jax.experimental import pallas as pl
from jax.experimental.pallas import tpu as pltpu
from jax.experimental.pallas import tpu_sc as plsc

B = 4096
S = 16
D = 512
E = 1024
C = 40
NNODES = 50000

NC = 2
NS = 16
NW = NC * NS
CH = D // 128
L = 16

SEG = 128
ZB = 32


def _sc_body(bpw, nk, acc, zps,
             nseg_hbm, ndst_hbm, sseg_hbm, feat_hbm, self_out, agg_out,
             nidx_v, ndst_v, sidx_v, rows_v, zeros_v, acc_sh, sem0, sem1):
    cid = lax.axis_index("c")
    sid = lax.axis_index("s")
    wid = cid * NS + sid
    base4 = pl.multiple_of(wid * (bpw * CH), bpw * CH)
    nb = pl.multiple_of(wid * (bpw * S * CH), bpw * S * CH)

    pltpu.sync_copy(nseg_hbm.at[pl.ds(nb, bpw * S * CH)], nidx_v)
    pltpu.sync_copy(ndst_hbm.at[pl.ds(nb, bpw * S * CH)], ndst_v)
    pltpu.sync_copy(sseg_hbm.at[pl.ds(base4, bpw * CH)], sidx_v)

    for t in range(ZB):
        for u in range(8):
            zeros_v[t, pl.ds(u * L, L)] = jnp.zeros((L,), jnp.float32)
    zbase = pl.multiple_of(sid * zps, zps)
    for m in range(zps // ZB):
        pltpu.sync_copy(zeros_v, acc_sh.at[pl.ds(zbase + m * ZB, ZB)])
    plsc.subcore_barrier()

    sems = (sem0, sem1)

    def nstart(k, slot):
        pltpu.async_copy(feat_hbm.at[nidx_v.at[pl.ds(k * SEG, SEG)]],
                         rows_v.at[pl.ds(slot * SEG, SEG)], sems[slot])

    def nwait(k, slot):
        pltpu.make_async_copy(feat_hbm.at[nidx_v.at[pl.ds(k * SEG, SEG)]],
                              rows_v.at[pl.ds(slot * SEG, SEG)], sems[slot]).wait()

    nstart(0, 0)
    nstart(1, 1)

    def body(i, carry):
        for half in range(2):
            k = 2 * i + half
            nwait(k, half)
            pltpu.sync_copy(rows_v.at[pl.ds(half * SEG, SEG)],
                            acc_sh.at[ndst_v.at[pl.ds(k * SEG, SEG)]],
                            add=True)

            @pl.when(k + 2 < nk)
            def _refill():
                nstart(k + 2, half)
        return carry

    lax.fori_loop(0, nk // 2, body, 0)

    nsp = bpw * CH // SEG

    def sstart(p, slot):
        pltpu.async_copy(feat_hbm.at[sidx_v.at[pl.ds(p * SEG, SEG)]],
                         rows_v.at[pl.ds(slot * SEG, SEG)], sems[slot])

    def swait(p, slot):
        pltpu.make_async_copy(feat_hbm.at[sidx_v.at[pl.ds(p * SEG, SEG)]],
                              rows_v.at[pl.ds(slot * SEG, SEG)], sems[slot]).wait()

    sstart(0, 0)
    if nsp > 1:
        sstart(1, 1)
    for p in range(nsp):
        slot = p % 2
        swait(p, slot)
        pltpu.sync_copy(rows_v.at[pl.ds(slot * SEG, SEG)],
                        self_out.at[pl.ds(base4 + p * SEG, SEG)])
        if p + 2 < nsp:
            sstart(p + 2, slot)

    plsc.subcore_barrier()
    for m in range(zps // SEG):
        off = zbase + m * SEG
        pltpu.sync_copy(acc_sh.at[pl.ds(off, SEG)],
                        agg_out.at[pl.ds(cid * acc + off, SEG)])


@functools.cache
def _make_sc_gather(b):
    bpw = b // NW
    nk = bpw * S * CH // SEG
    acc = b * CH // NC
    zps = acc // NS
    body = functools.partial(_sc_body, bpw, nk, acc, zps)
    return pl.kernel(
        body,
        out_type=[
            jax.ShapeDtypeStruct((b * CH, 128), jnp.float32),
            jax.ShapeDtypeStruct((b * CH, 128), jnp.float32),
        ],
        mesh=plsc.VectorSubcoreMesh(core_axis_name="c", subcore_axis_name="s",
                                    num_cores=NC, num_subcores=NS),
        scratch_types=[
            pltpu.VMEM((bpw * S * CH,), jnp.int32),
            pltpu.VMEM((bpw * S * CH,), jnp.int32),
            pltpu.VMEM((bpw * CH,), jnp.int32),
            pltpu.VMEM((2 * SEG, 128), jnp.float32),
            pltpu.VMEM((ZB, 128), jnp.float32),
            pltpu.VMEM_SHARED((acc, 128), jnp.float32),
            pltpu.SemaphoreType.DMA,
            pltpu.SemaphoreType.DMA,
        ],
        compiler_params=pltpu.CompilerParams(use_tc_tiling_on_sc=False),
    )


def _tc_body(self_ref, agg_ref, w1_ref, w2_ref, wcls_ref, out_ref):
    bb = self_ref.shape[0] // CH

    def logical(ref):
        x = ref[...].reshape(bb // 8, CH, 8, 128)
        return x.transpose(0, 2, 1, 3).reshape(bb, D)

    h = lax.dot_general(logical(self_ref), w1_ref[...],
                        (((1,), (1,)), ((), ())),
                        preferred_element_type=jnp.float32)
    h = h + lax.dot_general(logical(agg_ref), w2_ref[...],
                            (((1,), (1,)), ((), ())),
                            preferred_element_type=jnp.float32)
    h = jnp.maximum(h, 0.0)
    out_ref[...] = lax.dot_general(h, wcls_ref[...],
                                   (((1,), (1,)), ((), ())),
                                   preferred_element_type=jnp.float32)


def _tc_head(self2, agg2, w1, w2, wcls, b, block_b=512):
    grid = (b // block_b,)
    return pl.pallas_call(
        _tc_body,
        grid=grid,
        in_specs=[
            pl.BlockSpec((block_b * CH, 128), lambda i: (i, 0)),
            pl.BlockSpec((block_b * CH, 128), lambda i: (i, 0)),
            pl.BlockSpec((E, D), lambda i: (0, 0)),
            pl.BlockSpec((E, D), lambda i: (0, 0)),
            pl.BlockSpec((C, E), lambda i: (0, 0)),
        ],
        out_specs=pl.BlockSpec((block_b, C), lambda i: (i, 0)),
        out_shape=jax.ShapeDtypeStruct((b, C), jnp.float32),
    )(self2, agg2, w1, w2, wcls)


def _gather_indices(nodes, neigh_idx, b):
    j8 = jnp.arange(CH, dtype=jnp.int32) * 8
    cn = (neigh_idx >> 3) * 32 + (neigh_idx & 7)
    nseg = (cn[:, None, :] + j8[None, :, None]).reshape(-1)
    cs = (nodes >> 3) * 32 + (nodes & 7)
    sseg = (cs.reshape(b // 8, 8)[:, None, :] + j8[None, :, None]).reshape(-1)

    acc = b * CH // NC
    g = jnp.arange(b, dtype=jnp.int32)
    gcr = (g >> 3) * 32 + (g & 7) - (2 * g // b) * acc
    ndst = jnp.broadcast_to((gcr[:, None] + j8[None, :])[:, :, None],
                            (b, CH, S)).reshape(-1)
    return nseg, ndst, sseg


def kernel(nodes, neigh_idx, features, W_enc, weight):
    nodes = nodes.astype(jnp.int32)
    neigh_idx = neigh_idx.astype(jnp.int32)

    feat2 = (features.reshape(NNODES // 8, 8, CH, 128)
             .transpose(0, 2, 1, 3).reshape(NNODES * CH, 128))

    w1 = W_enc[:, :D]
    w2 = W_enc[:, D:] * (1.0 / S)

    hb = B // 2
    sc = _make_sc_gather(hb)
    outs = []
    for h in range(2):
        nh = lax.slice_in_dim(nodes, h * hb, (h + 1) * hb)
        eh = lax.slice_in_dim(neigh_idx, h * hb, (h + 1) * hb)
        nseg, ndst, sseg = _gather_indices(nh, eh, hb)
        self2, agg2 = sc(nseg, ndst, sseg, feat2)
        outs.append(_tc_head(self2, agg2, w1, w2, weight, hb))
    return jnp.concatenate(outs, axis=0)
